# baseline (speedup 1.0000x reference)
import functools
import math

import numpy as np
import jax
import jax.numpy as jnp
from jax import lax
from jax.experimental import pallas as pl
from jax.experimental.pallas import tpu as pltpu
from jax.experimental.pallas import tpu_sc as plsc

_STEPS = 64
_RUNS = 4
_B, _LQ, _L, _D = 32, 4, 8192, 64
_NQ = _B * _LQ
_NCH = _NQ * _RUNS
_NW = 32
_CPW = _NCH // _NW
_SCALE = 1.0 / math.sqrt(_D)


def _tf_rounds(x0, x1, k1, k2):
    ks = [np.uint32(k1), np.uint32(k2), np.uint32(k1 ^ k2 ^ np.uint32(0x1BD11BDA))]
    rot = [(13, 15, 26, 6), (17, 29, 16, 24)]
    x0 = (x0 + ks[0]).astype(np.uint32)
    x1 = (x1 + ks[1]).astype(np.uint32)
    for i in range(5):
        for r in rot[i % 2]:
            x0 = (x0 + x1).astype(np.uint32)
            x1 = ((x1 << np.uint32(r)) | (x1 >> np.uint32(32 - r))).astype(np.uint32)
            x1 = x0 ^ x1
        x0 = (x0 + ks[(i + 1) % 3]).astype(np.uint32)
        x1 = (x1 + ks[(i + 2) % 3] + np.uint32(i + 1)).astype(np.uint32)
    return x0, x1


def _fold_in(key, data):
    return _tf_rounds(np.uint32(0), np.uint32(data), key[0], key[1])


def _split2(key):
    b1, b2 = _tf_rounds(np.array([0, 0], np.uint32),
                        np.array([0, 1], np.uint32), key[0], key[1])
    return (b1[0], b2[0]), (b1[1], b2[1])


def _random_bits(key, n):
    b1, b2 = _tf_rounds(np.zeros(n, np.uint32),
                        np.arange(n, dtype=np.uint32), key[0], key[1])
    return b1 ^ b2


def _build_consts():
    with np.errstate(over="ignore"):
        base = (np.uint32(0), np.uint32(1234))
        vidx = np.empty((_STEPS, _NCH), np.int32)
        zz = np.empty((_STEPS, _NCH), np.float32)
        for s in range(_STEPS):
            ks = _fold_in(base, s)
            _, k2 = _split2(_fold_in(ks, 0))
            vidx[s] = (_random_bits(k2, _NCH) % np.uint32(_L)).astype(np.int32)
            bits = _random_bits(_fold_in(ks, 1), _NCH)
            fb = (bits >> np.uint32(9)) | np.uint32(0x3F800000)
            zz[s] = fb.view(np.float32) - np.float32(1.0)
    seen = np.zeros((_NCH, _L), bool)
    w = np.zeros((_STEPS, _NCH), bool)
    ar = np.arange(_NCH)
    for s in range(_STEPS - 1, -1, -1):
        w[s] = ~seen[ar, vidx[s]]
        seen[ar, vidx[s]] = True
    batch_idx = np.repeat(np.repeat(np.arange(_B), _LQ), _RUNS)
    gidx = (batch_idx[None, :].astype(np.int64) * _L + vidx).astype(np.int32)
    gidx_cm = np.ascontiguousarray(gidx.T)
    gpair = (gidx_cm >> 1).reshape(-1)
    ho = ((gidx_cm & 1) * _D).astype(np.int32)
    zt = np.ascontiguousarray(np.where(w, zz, np.float32(2.0)).T)
    return gpair, ho, zt.astype(np.float32)


_GIDX, _HO, _ZT = _build_consts()

_mesh = plsc.VectorSubcoreMesh(core_axis_name="c", subcore_axis_name="s")


@functools.partial(
    pl.kernel,
    out_type=jax.ShapeDtypeStruct((_NQ, _D), jnp.float32),
    mesh=_mesh,
    compiler_params=pltpu.CompilerParams(needs_layout_passes=False),
    scratch_types=[
        pltpu.VMEM((128,), jnp.int32),
        pltpu.VMEM((_CPW, _STEPS), jnp.float32),
        pltpu.VMEM((_CPW, _STEPS), jnp.int32),
        pltpu.VMEM((4, _D), jnp.float32),
        pltpu.VMEM((128, 2 * _D), jnp.float32),
        pltpu.VMEM((128, 2 * _D), jnp.float32),
        pltpu.VMEM((4, _D), jnp.float32),
        pltpu.SemaphoreType.DMA,
        pltpu.SemaphoreType.DMA,
    ],
)
def _sc_attn(qf, gidx, ho, zt, kf, vf, out, idx_v, zt_v, ho_v, qv, k_buf,
             v_buf, out_buf, sem_k, sem_v):
    wid = lax.axis_index("s") * 2 + lax.axis_index("c")
    base_ch = wid * _CPW
    pltpu.sync_copy(zt.at[pl.ds(base_ch, _CPW)], zt_v)
    pltpu.sync_copy(ho.at[pl.ds(base_ch, _CPW)], ho_v)
    pltpu.sync_copy(qf.at[pl.ds(wid * 4, 4)], qv)
    zero16 = jnp.zeros((16,), jnp.float32)
    for r in range(4):
        for u in range(4):
            out_buf[r, pl.ds(u * 16, 16)] = zero16
    iota = lax.iota(jnp.int32, 16)

    def blk_body(blk, carry):
        off = pl.multiple_of((wid * 8 + blk) * 128, 128)
        pltpu.sync_copy(gidx.at[pl.ds(off, 128)], idx_v)
        ck = pltpu.async_copy(kf.at[idx_v], k_buf, sem_k)
        cv = pltpu.async_copy(vf.at[idx_v], v_buf, sem_v)
        ck.wait()
        cv.wait()
        for c2 in range(2):
            ch = blk * 2 + c2
            qi = lax.shift_right_logical(ch, 2)

            def g_body(g, carry, c2=c2, ch=ch, qi=qi):
                a0, a1, a2, a3, cntv = carry
                row0 = c2 * 64 + g * 16
                rows = iota + row0
                hov = ho_v[ch, pl.ds(g * 16, 16)]

                def dot_body(i2, acc, rows=rows, hov=hov, qi=qi):
                    qvec = qv[qi, pl.ds(i2 * 16, 16)]
                    for u in range(16):
                        col = hov + (i2 * 16 + u)
                        kvv = plsc.load_gather(k_buf, [rows, col])
                        acc = acc + kvv * qvec[u]
                    return acc

                acc = lax.fori_loop(0, 4, dot_body,
                                    jnp.zeros((16,), jnp.float32))
                p = 1.0 / (1.0 + jnp.exp(acc * (-_SCALE)))
                ztg = zt_v[ch, pl.ds(g * 16, 16)]
                sel = jnp.where(ztg <= p, jnp.float32(1.0), jnp.float32(0.0))
                cntv = cntv + sel
                for u in range(16):
                    wj = sel[u]
                    hu = hov[u]
                    r = row0 + u
                    a0 = a0 + v_buf[r, pl.ds(hu, 16)] * wj
                    a1 = a1 + v_buf[r, pl.ds(hu + 16, 16)] * wj
                    a2 = a2 + v_buf[r, pl.ds(hu + 32, 16)] * wj
                    a3 = a3 + v_buf[r, pl.ds(hu + 48, 16)] * wj
                return (a0, a1, a2, a3, cntv)

            a0, a1, a2, a3, cntv = lax.fori_loop(
                0, 4, g_body, (zero16, zero16, zero16, zero16, zero16))
            cnt = jnp.sum(cntv)
            s = 0.25 / jnp.maximum(jnp.full((16,), cnt, jnp.float32), 1.0)
            for u, au in enumerate((a0, a1, a2, a3)):
                cur = out_buf[qi, pl.ds(u * 16, 16)]
                out_buf[qi, pl.ds(u * 16, 16)] = cur + au * s
        return carry

    lax.fori_loop(0, 8, blk_body, 0)
    pltpu.sync_copy(out_buf, out.at[pl.ds(wid * 4, 4)])


def kernel(q, k, v):
    B, Lq, d = q.shape
    qf = q.reshape(B * Lq, d)
    kf = k.reshape(-1, 2 * d)
    vf = v.reshape(-1, 2 * d)
    out = _sc_attn(qf, jnp.asarray(_GIDX), jnp.asarray(_HO), jnp.asarray(_ZT),
                   kf, vf)
    return out.reshape(B, Lq, d)


# speedup vs baseline: 3.3859x; 3.3859x over previous
import functools
import math

import numpy as np
import jax
import jax.numpy as jnp
from jax import lax
from jax.experimental import pallas as pl
from jax.experimental.pallas import tpu as pltpu
from jax.experimental.pallas import tpu_sc as plsc

_STEPS = 64
_RUNS = 4
_B, _LQ, _L, _D = 32, 4, 8192, 64
_NQ = _B * _LQ
_NCH = _NQ * _RUNS
_NW = 32
_CPW = _NCH // _NW
_SCALE = 1.0 / math.sqrt(_D)
_NBUF = 4


def _tf_rounds(x0, x1, k1, k2):
    ks = [np.uint32(k1), np.uint32(k2), np.uint32(k1 ^ k2 ^ np.uint32(0x1BD11BDA))]
    rot = [(13, 15, 26, 6), (17, 29, 16, 24)]
    x0 = (x0 + ks[0]).astype(np.uint32)
    x1 = (x1 + ks[1]).astype(np.uint32)
    for i in range(5):
        for r in rot[i % 2]:
            x0 = (x0 + x1).astype(np.uint32)
            x1 = ((x1 << np.uint32(r)) | (x1 >> np.uint32(32 - r))).astype(np.uint32)
            x1 = x0 ^ x1
        x0 = (x0 + ks[(i + 1) % 3]).astype(np.uint32)
        x1 = (x1 + ks[(i + 2) % 3] + np.uint32(i + 1)).astype(np.uint32)
    return x0, x1


def _fold_in(key, data):
    return _tf_rounds(np.uint32(0), np.uint32(data), key[0], key[1])


def _split2(key):
    b1, b2 = _tf_rounds(np.array([0, 0], np.uint32),
                        np.array([0, 1], np.uint32), key[0], key[1])
    return (b1[0], b2[0]), (b1[1], b2[1])


def _random_bits(key, n):
    b1, b2 = _tf_rounds(np.zeros(n, np.uint32),
                        np.arange(n, dtype=np.uint32), key[0], key[1])
    return b1 ^ b2


def _build_consts():
    with np.errstate(over="ignore"):
        base = (np.uint32(0), np.uint32(1234))
        vidx = np.empty((_STEPS, _NCH), np.int32)
        zz = np.empty((_STEPS, _NCH), np.float32)
        for s in range(_STEPS):
            ks = _fold_in(base, s)
            _, k2 = _split2(_fold_in(ks, 0))
            vidx[s] = (_random_bits(k2, _NCH) % np.uint32(_L)).astype(np.int32)
            bits = _random_bits(_fold_in(ks, 1), _NCH)
            fb = (bits >> np.uint32(9)) | np.uint32(0x3F800000)
            zz[s] = fb.view(np.float32) - np.float32(1.0)
    seen = np.zeros((_NCH, _L), bool)
    w = np.zeros((_STEPS, _NCH), bool)
    ar = np.arange(_NCH)
    for s in range(_STEPS - 1, -1, -1):
        w[s] = ~seen[ar, vidx[s]]
        seen[ar, vidx[s]] = True
    vidx_cm = np.ascontiguousarray(vidx.T)
    zt = np.ascontiguousarray(np.where(w, zz, np.float32(2.0)).T)
    return vidx_cm, zt.astype(np.float32)


_VIDX, _ZT = _build_consts()

_mesh = plsc.VectorSubcoreMesh(core_axis_name="c", subcore_axis_name="s")


@functools.partial(
    pl.kernel,
    out_type=jax.ShapeDtypeStruct((_NQ, _D), jnp.float32),
    mesh=_mesh,
    compiler_params=pltpu.CompilerParams(needs_layout_passes=False),
    scratch_types=[
        pltpu.VMEM((_CPW, _STEPS), jnp.int32),
        pltpu.VMEM((_CPW, _STEPS), jnp.float32),
        pltpu.VMEM((4, _D), jnp.float32),
        [pltpu.VMEM((1, _L), jnp.float32) for _ in range(_NBUF)],
        pltpu.VMEM((_CPW, _STEPS), jnp.float32),
        pltpu.VMEM((_STEPS, _CPW), jnp.int32),
        pltpu.VMEM((_STEPS, _CPW), jnp.float32),
        pltpu.VMEM((_STEPS, _CPW), jnp.float32),
        pltpu.VMEM((4, _D), jnp.float32),
        [pltpu.SemaphoreType.DMA for _ in range(_NBUF)],
    ],
)
def _sc_attn(qf, jc, zt, kt, vt, out, jv, zt_v, qv, ring, acc_cm, jT, selT,
             out_cm, out_buf, sems):
    wid = lax.axis_index("s") * 2 + lax.axis_index("c")
    base_ch = wid * _CPW
    pltpu.sync_copy(jc.at[pl.ds(base_ch, _CPW)], jv)
    pltpu.sync_copy(zt.at[pl.ds(base_ch, _CPW)], zt_v)
    pltpu.sync_copy(qf.at[pl.ds(wid * 4, 4)], qv)
    zero16 = jnp.zeros((16,), jnp.float32)
    iota = lax.iota(jnp.int32, 16)
    row0 = wid * _D

    for c in range(_CPW):
        for g in range(4):
            acc_cm[c, pl.ds(g * 16, 16)] = zero16

    for u in range(_NBUF):
        pltpu.async_copy(kt.at[pl.ds(row0 + u, 1)], ring[u], sems[u])

    def k_body(i4, carry):
        for u in range(_NBUF):
            i = i4 * _NBUF + u
            pltpu.make_async_copy(kt.at[pl.ds(row0 + i, 1)], ring[u],
                                  sems[u]).wait()
            qcol = plsc.load_gather(
                qv, [lax.shift_right_logical(iota, 2), jnp.full((16,), i, jnp.int32)])
            for c in range(_CPW):
                qs = qcol[c]
                for g in range(4):
                    jl = jv[c, pl.ds(g * 16, 16)]
                    vals = plsc.load_gather(ring[u], [jnp.zeros((16,), jnp.int32), jl])
                    a = acc_cm[c, pl.ds(g * 16, 16)]
                    acc_cm[c, pl.ds(g * 16, 16)] = a + vals * qs
            nxt = i + _NBUF

            @pl.when(nxt < _D)
            def _():
                pltpu.async_copy(kt.at[pl.ds(row0 + nxt, 1)], ring[u], sems[u])
        return carry

    lax.fori_loop(0, _D // _NBUF, k_body, 0)

    for c in range(_CPW):
        for g in range(4):
            a = acc_cm[c, pl.ds(g * 16, 16)]
            p = 1.0 / (1.0 + jnp.exp(a * (-_SCALE)))
            ztg = zt_v[c, pl.ds(g * 16, 16)]
            acc_cm[c, pl.ds(g * 16, 16)] = jnp.where(
                ztg <= p, jnp.float32(1.0), jnp.float32(0.0))
    for s in range(_STEPS):
        scol = jnp.full((16,), s, jnp.int32)
        selT[s, pl.ds(0, 16)] = plsc.load_gather(acc_cm, [iota, scol])
        jT[s, pl.ds(0, 16)] = plsc.load_gather(jv, [iota, scol])

    cnt = zero16
    for s in range(_STEPS):
        cnt = cnt + selT[s, pl.ds(0, 16)]
    sc16 = 0.25 / jnp.maximum(cnt, 1.0)

    for u in range(_NBUF):
        pltpu.async_copy(vt.at[pl.ds(row0 + u, 1)], ring[u], sems[u])

    def v_body(i4, carry):
        for u in range(_NBUF):
            i = i4 * _NBUF + u
            pltpu.make_async_copy(vt.at[pl.ds(row0 + i, 1)], ring[u],
                                  sems[u]).wait()
            acc = zero16
            for s in range(_STEPS):
                vals = plsc.load_gather(
                    ring[u], [jnp.zeros((16,), jnp.int32), jT[s, pl.ds(0, 16)]])
                acc = acc + vals * selT[s, pl.ds(0, 16)]
            out_cm[i, pl.ds(0, 16)] = acc * sc16
            nxt = i + _NBUF

            @pl.when(nxt < _D)
            def _():
                pltpu.async_copy(vt.at[pl.ds(row0 + nxt, 1)], ring[u], sems[u])
        return carry

    lax.fori_loop(0, _D // _NBUF, v_body, 0)

    for qi in range(4):
        for g in range(4):
            rows = iota + g * 16
            o = zero16
            for run in range(4):
                col = jnp.full((16,), qi * 4 + run, jnp.int32)
                o = o + plsc.load_gather(out_cm, [rows, col])
            out_buf[qi, pl.ds(g * 16, 16)] = o
    pltpu.sync_copy(out_buf, out.at[pl.ds(wid * 4, 4)])


def kernel(q, k, v):
    B, Lq, d = q.shape
    L = k.shape[1]
    qf = q.reshape(B * Lq, d)
    kt = k.transpose(0, 2, 1).reshape(B * d, L)
    vt = v.transpose(0, 2, 1).reshape(B * d, L)
    out = _sc_attn(qf, jnp.asarray(_VIDX), jnp.asarray(_ZT), kt, vt)
    return out.reshape(B, Lq, d)


# speedup vs baseline: 3.6770x; 1.0860x over previous
import functools
import math

import numpy as np
import jax
import jax.numpy as jnp
from jax import lax
from jax.experimental import pallas as pl
from jax.experimental.pallas import tpu as pltpu
from jax.experimental.pallas import tpu_sc as plsc

_STEPS = 64
_RUNS = 4
_B, _LQ, _L, _D = 32, 4, 8192, 64
_NQ = _B * _LQ
_NCH = _NQ * _RUNS
_CPW = 16
_SCALE = 1.0 / math.sqrt(_D)
_NBUF = 4
_NSC = 16
_HD = _D // 2


def _tf_rounds(x0, x1, k1, k2):
    ks = [np.uint32(k1), np.uint32(k2), np.uint32(k1 ^ k2 ^ np.uint32(0x1BD11BDA))]
    rot = [(13, 15, 26, 6), (17, 29, 16, 24)]
    x0 = (x0 + ks[0]).astype(np.uint32)
    x1 = (x1 + ks[1]).astype(np.uint32)
    for i in range(5):
        for r in rot[i % 2]:
            x0 = (x0 + x1).astype(np.uint32)
            x1 = ((x1 << np.uint32(r)) | (x1 >> np.uint32(32 - r))).astype(np.uint32)
            x1 = x0 ^ x1
        x0 = (x0 + ks[(i + 1) % 3]).astype(np.uint32)
        x1 = (x1 + ks[(i + 2) % 3] + np.uint32(i + 1)).astype(np.uint32)
    return x0, x1


def _fold_in(key, data):
    return _tf_rounds(np.uint32(0), np.uint32(data), key[0], key[1])


def _split2(key):
    b1, b2 = _tf_rounds(np.array([0, 0], np.uint32),
                        np.array([0, 1], np.uint32), key[0], key[1])
    return (b1[0], b2[0]), (b1[1], b2[1])


def _random_bits(key, n):
    b1, b2 = _tf_rounds(np.zeros(n, np.uint32),
                        np.arange(n, dtype=np.uint32), key[0], key[1])
    return b1 ^ b2


def _build_consts():
    with np.errstate(over="ignore"):
        base = (np.uint32(0), np.uint32(1234))
        vidx = np.empty((_STEPS, _NCH), np.int32)
        zz = np.empty((_STEPS, _NCH), np.float32)
        for s in range(_STEPS):
            ks = _fold_in(base, s)
            _, k2 = _split2(_fold_in(ks, 0))
            vidx[s] = (_random_bits(k2, _NCH) % np.uint32(_L)).astype(np.int32)
            bits = _random_bits(_fold_in(ks, 1), _NCH)
            fb = (bits >> np.uint32(9)) | np.uint32(0x3F800000)
            zz[s] = fb.view(np.float32) - np.float32(1.0)
    seen = np.zeros((_NCH, _L), bool)
    w = np.zeros((_STEPS, _NCH), bool)
    ar = np.arange(_NCH)
    for s in range(_STEPS - 1, -1, -1):
        w[s] = ~seen[ar, vidx[s]]
        seen[ar, vidx[s]] = True
    vidx_cm = np.ascontiguousarray(vidx.T)
    zt_cm = np.ascontiguousarray(np.where(w, zz, np.float32(2.0)).T)
    td = np.full((_NCH, _L), 2.0, np.float32)
    for s in range(_STEPS):
        td[ar, vidx[s]] = zz[s]
    return vidx_cm, zt_cm.astype(np.float32), td


_VIDX, _ZT, _TD = _build_consts()

_mesh = plsc.VectorSubcoreMesh(core_axis_name="c", subcore_axis_name="s")


@functools.partial(
    pl.kernel,
    out_type=jax.ShapeDtypeStruct((2, _NSC * _LQ, _HD), jnp.float32),
    mesh=_mesh,
    compiler_params=pltpu.CompilerParams(needs_layout_passes=False),
    scratch_types=[
        pltpu.VMEM((_CPW, _STEPS), jnp.int32),
        pltpu.VMEM((_CPW, _STEPS), jnp.float32),
        pltpu.VMEM((4, _D), jnp.float32),
        [pltpu.VMEM((1, _L), jnp.float32) for _ in range(_NBUF)],
        pltpu.VMEM((_CPW, _STEPS), jnp.float32),
        pltpu.VMEM((_STEPS, _CPW), jnp.int32),
        pltpu.VMEM((_STEPS, _CPW), jnp.float32),
        pltpu.VMEM((_HD, _CPW), jnp.float32),
        pltpu.VMEM((4, _HD), jnp.float32),
        [pltpu.SemaphoreType.DMA for _ in range(_NBUF)],
    ],
)
def _sc_attn(qf, jc, zt, kt, vt, out, jv, zt_v, qv, ring, acc_cm, jT, selT,
             out_cm, out_buf, sems):
    wid = lax.axis_index("s") * 2 + lax.axis_index("c")
    b = lax.bitwise_and(wid, 15)
    fh = lax.shift_right_logical(wid, 4)
    base_ch = b * _CPW
    pltpu.sync_copy(jc.at[pl.ds(base_ch, _CPW)], jv)
    pltpu.sync_copy(zt.at[pl.ds(base_ch, _CPW)], zt_v)
    pltpu.sync_copy(qf.at[pl.ds(b * 4, 4)], qv)
    zero16 = jnp.zeros((16,), jnp.float32)
    iota = lax.iota(jnp.int32, 16)
    row0 = b * _D

    for c in range(_CPW):
        for g in range(4):
            acc_cm[c, pl.ds(g * 16, 16)] = zero16

    for u in range(_NBUF):
        pltpu.async_copy(kt.at[pl.ds(row0 + u, 1)], ring[u], sems[u])

    def k_body(i4, carry):
        for u in range(_NBUF):
            i = i4 * _NBUF + u
            pltpu.make_async_copy(kt.at[pl.ds(row0 + i, 1)], ring[u],
                                  sems[u]).wait()
            qcol = plsc.load_gather(
                qv, [lax.shift_right_logical(iota, 2),
                     jnp.full((16,), i, jnp.int32)])
            for c in range(_CPW):
                qs = qcol[c]
                for g in range(4):
                    jl = jv[c, pl.ds(g * 16, 16)]
                    vals = plsc.load_gather(
                        ring[u], [jnp.zeros((16,), jnp.int32), jl])
                    a = acc_cm[c, pl.ds(g * 16, 16)]
                    acc_cm[c, pl.ds(g * 16, 16)] = a + vals * qs
            nxt = i + _NBUF

            @pl.when(nxt < _D)
            def _():
                pltpu.async_copy(kt.at[pl.ds(row0 + nxt, 1)], ring[u], sems[u])
        return carry

    lax.fori_loop(0, _D // _NBUF, k_body, 0)

    for c in range(_CPW):
        for g in range(4):
            a = acc_cm[c, pl.ds(g * 16, 16)]
            p = 1.0 / (1.0 + jnp.exp(a * (-_SCALE)))
            ztg = zt_v[c, pl.ds(g * 16, 16)]
            acc_cm[c, pl.ds(g * 16, 16)] = jnp.where(
                ztg <= p, jnp.float32(1.0), jnp.float32(0.0))
    for s in range(_STEPS):
        scol = jnp.full((16,), s, jnp.int32)
        selT[s, pl.ds(0, 16)] = plsc.load_gather(acc_cm, [iota, scol])
        jT[s, pl.ds(0, 16)] = plsc.load_gather(jv, [iota, scol])

    cnt = zero16
    for s in range(_STEPS):
        cnt = cnt + selT[s, pl.ds(0, 16)]
    sc16 = 0.25 / jnp.maximum(cnt, 1.0)

    vrow0 = row0 + fh * _HD
    for u in range(_NBUF):
        pltpu.async_copy(vt.at[pl.ds(vrow0 + u, 1)], ring[u], sems[u])

    def v_body(i4, carry):
        for u in range(_NBUF):
            i = i4 * _NBUF + u
            pltpu.make_async_copy(vt.at[pl.ds(vrow0 + i, 1)], ring[u],
                                  sems[u]).wait()
            acc = zero16
            for s in range(_STEPS):
                vals = plsc.load_gather(
                    ring[u], [jnp.zeros((16,), jnp.int32), jT[s, pl.ds(0, 16)]])
                acc = acc + vals * selT[s, pl.ds(0, 16)]
            out_cm[i, pl.ds(0, 16)] = acc * sc16
            nxt = i + _NBUF

            @pl.when(nxt < _HD)
            def _():
                pltpu.async_copy(vt.at[pl.ds(vrow0 + nxt, 1)], ring[u], sems[u])
        return carry

    lax.fori_loop(0, _HD // _NBUF, v_body, 0)

    for qi in range(4):
        for g in range(_HD // 16):
            rows = iota + g * 16
            o = zero16
            for run in range(4):
                col = jnp.full((16,), qi * 4 + run, jnp.int32)
                o = o + plsc.load_gather(out_cm, [rows, col])
            out_buf[qi, pl.ds(g * 16, 16)] = o
    pltpu.sync_copy(out_buf, out.at[fh, pl.ds(b * 4, 4)])


def _tc_body(q_ref, k_ref, v_ref, td_ref, o_ref):
    qm = q_ref[0]
    s = jnp.dot(qm, k_ref[...], preferred_element_type=jnp.float32,
                precision=lax.Precision.HIGHEST)
    p = 1.0 / (1.0 + jnp.exp(s * (-_SCALE)))
    p16 = jnp.broadcast_to(p[:, None, :], (4, 4, _L)).reshape(16, _L)
    sel = jnp.where(td_ref[...] <= p16, jnp.float32(1.0), jnp.float32(0.0))
    cnt = jnp.sum(sel, axis=1)
    sv = lax.dot_general(sel, v_ref[...], (((1,), (1,)), ((), ())),
                         preferred_element_type=jnp.float32,
                         precision=lax.Precision.HIGHEST)
    oc = sv * (0.25 / jnp.maximum(cnt, 1.0))[:, None]
    o_ref[0] = oc.reshape(4, 4, _D).sum(axis=1)


_tc_attn = pl.pallas_call(
    _tc_body,
    grid=(_B - _NSC,),
    in_specs=[
        pl.BlockSpec((1, _LQ, _D), lambda b: (b + _NSC, 0, 0)),
        pl.BlockSpec((_D, _L), lambda b: (b + _NSC, 0)),
        pl.BlockSpec((_D, _L), lambda b: (b + _NSC, 0)),
        pl.BlockSpec((_CPW, _L), lambda b: (b + _NSC, 0)),
    ],
    out_specs=pl.BlockSpec((1, _LQ, _D), lambda b: (b, 0, 0)),
    out_shape=jax.ShapeDtypeStruct((_B - _NSC, _LQ, _D), jnp.float32),
)


def kernel(q, k, v):
    B, Lq, d = q.shape
    L = k.shape[1]
    qf = q.reshape(B * Lq, d)
    kt = k.transpose(0, 2, 1).reshape(B * d, L)
    vt = v.transpose(0, 2, 1).reshape(B * d, L)
    out_sc = _sc_attn(qf, jnp.asarray(_VIDX), jnp.asarray(_ZT), kt, vt)
    out_tc = _tc_attn(q, kt, vt, jnp.asarray(_TD))
    sc = jnp.concatenate([out_sc[0], out_sc[1]], axis=-1)
    out = jnp.concatenate([sc.reshape(_NSC, Lq, d), out_tc], axis=0)
    return out
